# Initial kernel scaffold; baseline (speedup 1.0000x reference)
import jax
import jax.numpy as jnp
from jax.experimental import pallas as pl


def kernel(seqs, targets, labels, table, W, b):
    raise NotImplementedError("write your pallas kernel here")


# speedup vs baseline: 1.6839x; 1.6839x over previous
import functools

import jax
import jax.numpy as jnp
from jax import lax
from jax.experimental import pallas as pl
from jax.experimental.pallas import tpu as pltpu
from jax.experimental.pallas import tpu_sc as plsc

VOCAB = 1000000
H = 64
S = 200
B = 4096
T = 4
L = 16
NC = 2
NS = 16
NW = NC * NS
BPW = B // NW
NCHUNK = BPW // L
ROWS = T * BPW


def _sc_body(seqs_hbm, tgt_hbm, table_hbm, w_hbm, out_hbm,
             idx_v, tok_v, rows_v, w_v, wbc_v, pred_v, sem):
    wid = lax.axis_index("s") * NC + lax.axis_index("c")
    base = wid * BPW

    for t in range(T):
        pltpu.sync_copy(tgt_hbm.at[pl.ds(t * B + base, BPW)], idx_v.at[t])
    pltpu.sync_copy(w_hbm, w_v)
    for t in range(T):
        for j in range(NCHUNK):
            col = lax.iota(jnp.int32, L) + (base + j * L)
            sl = pl.ds(j * L, L)
            idx_v[t, sl] = idx_v[t, sl] * B + col

    cps = [pltpu.async_copy(seqs_hbm.at[idx_v.at[t]], tok_v.at[t], sem)
           for t in range(T)]
    for cp in cps:
        cp.wait()

    def wfill(j, carry):
        wbc_v[j, :] = plsc.load_gather(w_v, [jnp.full((L,), j, jnp.int32)])
        return carry
    lax.fori_loop(0, T * H, wfill, 0)

    def fetch(k, carry):
        t = k // NCHUNK
        tokc = tok_v[t, pl.ds((k % NCHUNK) * L, L)]
        for l in range(L):
            pltpu.async_copy(table_hbm.at[tokc[l]], rows_v.at[k * L + l], sem)
        return carry
    lax.fori_loop(0, T * NCHUNK, fetch, 0)
    pltpu.make_async_copy(table_hbm.at[pl.ds(0, ROWS)], rows_v, sem).wait()

    for t in range(T):
        iidx = [lax.iota(jnp.int32, L) + (t * BPW + c * L)
                for c in range(NCHUNK)]

        def hbody(h, accs, t=t, iidx=iidx):
            bw = wbc_v[t * H + h, :]
            colh = jnp.full((L,), h, jnp.int32)
            return tuple(
                accs[c] + plsc.load_gather(rows_v, [iidx[c], colh]) * bw
                for c in range(NCHUNK))

        accs = lax.fori_loop(
            0, H, hbody, tuple(jnp.zeros((L,), jnp.float32)
                               for _ in range(NCHUNK)))
        for c in range(NCHUNK):
            sl = pl.ds(c * L, L)
            if t == 0:
                pred_v[sl] = accs[c]
            else:
                pred_v[sl] = pred_v[sl] + accs[c]

    pltpu.sync_copy(pred_v, out_hbm.at[pl.ds(base, BPW)])


_sc_gather = functools.partial(
    pl.kernel,
    out_type=jax.ShapeDtypeStruct((B,), jnp.float32),
    mesh=plsc.VectorSubcoreMesh(core_axis_name="c", subcore_axis_name="s"),
    compiler_params=pltpu.CompilerParams(needs_layout_passes=False),
    scratch_types=[
        pltpu.VMEM((T, BPW), jnp.int32),
        pltpu.VMEM((T, BPW), jnp.int32),
        pltpu.VMEM((ROWS, H), jnp.float32),
        pltpu.VMEM((T * H,), jnp.float32),
        pltpu.VMEM((T * H, L), jnp.float32),
        pltpu.VMEM((BPW,), jnp.float32),
        pltpu.SemaphoreType.DMA,
    ],
)(_sc_body)


def _loss_body(pred_ref, lab_ref, b_ref, out_ref):
    p = pred_ref[:] + b_ref[0]
    lab = lab_ref[:]
    terms = (jnp.maximum(p, 0.0) - p * lab
             + jnp.log(1.0 + jnp.exp(-jnp.abs(p))))
    out_ref[0, 0] = jnp.sum(terms) * (1.0 / B)


_loss_call = pl.pallas_call(
    _loss_body,
    out_shape=jax.ShapeDtypeStruct((1, 1), jnp.float32),
    in_specs=[
        pl.BlockSpec(memory_space=pltpu.VMEM),
        pl.BlockSpec(memory_space=pltpu.VMEM),
        pl.BlockSpec(memory_space=pltpu.SMEM),
    ],
    out_specs=pl.BlockSpec(memory_space=pltpu.SMEM),
)


def kernel(seqs, targets, labels, table, W, b):
    seqs_flat = seqs.reshape(-1).astype(jnp.int32)
    tgt_flat = targets.reshape(-1).astype(jnp.int32)
    w_flat = W.reshape(-1)
    pred = _sc_gather(seqs_flat, tgt_flat, table, w_flat)
    loss = _loss_call(pred.reshape(B // 128, 128),
                      labels.reshape(B // 128, 128), b)
    return loss[0, 0]
